# Initial kernel scaffold; baseline (speedup 1.0000x reference)
import jax
import jax.numpy as jnp
from jax.experimental import pallas as pl


def kernel(x, edge_index, W0, b0, W1, b1, W2, b2):
    raise NotImplementedError("write your pallas kernel here")


# speedup vs baseline: 19.8367x; 19.8367x over previous
import functools

import jax
import jax.numpy as jnp
from jax import lax
from jax.experimental import pallas as pl
from jax.experimental.pallas import tpu as pltpu
from jax.experimental.pallas import tpu_sc as plsc

N = 10000
E = 320000
D_IN = 128
D_HID = 128
D_OUT = 40
DH = 64

NC, NS, LANES = 2, 16, 16
NW = NC * NS
B = 125
NBLK = 80
ACC_ROWS = 10240
RPT = ACC_ROWS // NS
ZR = 64

_MESH = dict(core_axis_name="c", subcore_axis_name="s",
             num_cores=NC, num_subcores=NS)


def _make_sc_scatter(nh):
    @functools.partial(
        pl.kernel,
        out_type=jax.ShapeDtypeStruct((NC, nh, ACC_ROWS, DH), jnp.float32),
        mesh=plsc.VectorSubcoreMesh(**_MESH),
        compiler_params=pltpu.CompilerParams(use_tc_tiling_on_sc=False),
        scratch_types=[
            pltpu.VMEM((NBLK, B), jnp.int32),
            pltpu.VMEM((NBLK, B), jnp.int32),
            pltpu.VMEM((2, B, DH), jnp.float32),
            pltpu.VMEM((ZR, DH), jnp.float32),
            pltpu.VMEM_SHARED((ACC_ROWS, DH), jnp.float32),
            pltpu.SemaphoreType.DMA,
            pltpu.SemaphoreType.DMA,
            pltpu.SemaphoreType.DMA,
            pltpu.SemaphoreType.DMA,
        ],
    )
    def sc_scatter(*refs):
        g_hbm = refs[:nh]
        srcb_hbm, dstb_hbm, out_hbm = refs[nh], refs[nh + 1], refs[nh + 2]
        idx_s, idx_d, rows, zbuf, acc, gs0, gs1, ss0, ss1 = refs[nh + 3:]

        c = lax.axis_index("c")
        s = lax.axis_index("s")
        wid = c * NS + s
        base = s * RPT

        pltpu.sync_copy(srcb_hbm.at[wid], idx_s)
        pltpu.sync_copy(dstb_hbm.at[wid], idx_d)

        zero = jnp.zeros((LANES,), jnp.float32)

        def zrow(i, carry):
            for j in range(DH // LANES):
                zbuf[i, pl.ds(j * LANES, LANES)] = zero
            return carry

        lax.fori_loop(0, ZR, zrow, 0)

        gsem = (gs0, gs1)
        ssem = (ss0, ss1)

        for h in range(nh):
            g = g_hbm[h]

            for t in range(RPT // ZR):
                pltpu.sync_copy(zbuf, acc.at[pl.ds(base + t * ZR, ZR)])
            plsc.subcore_barrier()

            def g_start(blk, b):
                pltpu.make_async_copy(g.at[idx_s.at[blk]], rows.at[b],
                                      gsem[b]).start()

            def g_wait(b):
                pltpu.make_async_copy(g.at[idx_s.at[0]], rows.at[b],
                                      gsem[b]).wait()

            def s_start(blk, b):
                pltpu.async_copy(rows.at[b], acc.at[idx_d.at[blk]],
                                 ssem[b], add=True)

            def s_wait(b):
                pltpu.make_async_copy(rows.at[b], acc.at[idx_d.at[0]],
                                      ssem[b]).wait()

            g_start(0, 0)

            def body(j, carry):
                e = 2 * j
                g_wait(0)

                @pl.when(j >= 1)
                def _():
                    s_wait(1)

                g_start(e + 1, 1)
                s_start(e, 0)
                g_wait(1)
                s_wait(0)

                @pl.when(j + 1 < NBLK // 2)
                def _():
                    g_start(e + 2, 0)

                s_start(e + 1, 1)
                return carry

            lax.fori_loop(0, NBLK // 2, body, 0)
            s_wait(1)
            plsc.subcore_barrier()
            pltpu.sync_copy(acc.at[pl.ds(base, RPT)],
                            out_hbm.at[c, h, pl.ds(base, RPT)])

    return sc_scatter


_sc_scatter2 = _make_sc_scatter(2)
_sc_scatter1 = _make_sc_scatter(1)


@functools.partial(
    pl.kernel,
    out_type=jax.ShapeDtypeStruct((NC, ACC_ROWS), jnp.float32),
    mesh=plsc.VectorSubcoreMesh(**_MESH),
    compiler_params=pltpu.CompilerParams(use_tc_tiling_on_sc=False),
    scratch_types=[
        pltpu.VMEM((NBLK, B), jnp.int32),
        pltpu.VMEM((128,), jnp.float32),
        pltpu.VMEM((RPT,), jnp.float32),
        pltpu.VMEM_SHARED((ACC_ROWS,), jnp.float32),
    ],
)
def _sc_degree(dstb_hbm, out_hbm, idx_d, ones, zbuf, acc):
    c = lax.axis_index("c")
    s = lax.axis_index("s")
    wid = c * NS + s

    pltpu.sync_copy(dstb_hbm.at[wid], idx_d)

    one = jnp.full((LANES,), 1.0, jnp.float32)
    for j in range(128 // LANES):
        ones[pl.ds(j * LANES, LANES)] = one
    zero = jnp.zeros((LANES,), jnp.float32)

    def zrow(i, carry):
        zbuf[pl.ds(i * LANES, LANES)] = zero
        return carry

    lax.fori_loop(0, RPT // LANES, zrow, 0)
    base = s * RPT
    pltpu.sync_copy(zbuf, acc.at[pl.ds(base, RPT)])
    plsc.subcore_barrier()

    def body(blk, carry):
        pltpu.sync_copy(ones.at[pl.ds(0, B)], acc.at[idx_d.at[blk]], add=True)
        return carry

    lax.fori_loop(0, NBLK, body, 0)
    plsc.subcore_barrier()
    pltpu.sync_copy(acc.at[pl.ds(base, RPT)], out_hbm.at[c, pl.ds(base, RPT)])


NP = ACC_ROWS
_BN = 1024
_GRID = NP // _BN

_HALF = pl.BlockSpec((_BN, DH), lambda i: (i, 0))
_FULL = pl.BlockSpec((_BN, D_HID), lambda i: (i, 0))
_COL1 = pl.BlockSpec((_BN, 1), lambda i: (i, 0))


def _tc_first(degt, x, W0):
    def body(deg_ref, x_ref, w_ref, gl_ref, gr_ref, dinv_ref):
        deg = deg_ref[0, :] + deg_ref[1, :] + 1.0
        dinv = lax.rsqrt(deg)
        y = jnp.dot(x_ref[...], w_ref[...], preferred_element_type=jnp.float32)
        g = y * dinv[:, None]
        gl_ref[...] = g[:, :DH]
        gr_ref[...] = g[:, DH:]
        dinv_ref[...] = dinv[:, None]

    return pl.pallas_call(
        body,
        grid=(_GRID,),
        in_specs=[
            pl.BlockSpec((NC, _BN), lambda i: (0, i)),
            pl.BlockSpec((_BN, D_IN), lambda i: (i, 0)),
            pl.BlockSpec((D_IN, D_HID), lambda i: (0, 0)),
        ],
        out_specs=[_HALF, _HALF, _COL1],
        out_shape=[
            jax.ShapeDtypeStruct((NP, DH), jnp.float32),
            jax.ShapeDtypeStruct((NP, DH), jnp.float32),
            jax.ShapeDtypeStruct((NP, 1), jnp.float32),
        ],
    )(degt, x, W0)


def _tc_mid(part, gl, gr, dinv, mask2, b_prev, W, n_out_halves):
    d_out = n_out_halves * DH

    def body(p_ref, gl_ref, gr_ref, dinv_ref, m_ref, b_ref, w_ref, *o_refs):
        dinv = dinv_ref[...]
        tl = dinv * (p_ref[0, 0] + p_ref[1, 0] + gl_ref[...]) + b_ref[:, :DH]
        tr = dinv * (p_ref[0, 1] + p_ref[1, 1] + gr_ref[...]) + b_ref[:, DH:]
        t = jnp.concatenate([tl, tr], axis=1)
        h = jnp.where(t > 0, t, jnp.exp(t) - 1.0) * m_ref[...]
        y = jnp.dot(h, w_ref[...], preferred_element_type=jnp.float32)
        g = y * dinv
        for k, o_ref in enumerate(o_refs):
            o_ref[...] = g[:, k * DH:(k + 1) * DH]

    return pl.pallas_call(
        body,
        grid=(_GRID,),
        in_specs=[
            pl.BlockSpec((NC, 2, _BN, DH), lambda i: (0, 0, i, 0)),
            _HALF, _HALF, _COL1, _FULL,
            pl.BlockSpec((1, D_HID), lambda i: (0, 0)),
            pl.BlockSpec((D_HID, d_out), lambda i: (0, 0)),
        ],
        out_specs=[_HALF] * n_out_halves,
        out_shape=[jax.ShapeDtypeStruct((NP, DH), jnp.float32)] * n_out_halves,
    )(part, gl, gr, dinv, mask2, b_prev, W)


def _tc_final(part, g2, dinv, b2p):
    def body(p_ref, g_ref, dinv_ref, b_ref, o_ref):
        t = dinv_ref[...] * (p_ref[0, 0] + p_ref[1, 0] + g_ref[...]) \
            + b_ref[...]
        m = jnp.max(t, axis=1, keepdims=True)
        lse = jnp.log(jnp.sum(jnp.exp(t - m), axis=1, keepdims=True)) + m
        o_ref[...] = t - lse

    return pl.pallas_call(
        body,
        grid=(_GRID,),
        in_specs=[
            pl.BlockSpec((NC, 1, _BN, DH), lambda i: (0, 0, i, 0)),
            _HALF, _COL1,
            pl.BlockSpec((1, DH), lambda i: (0, 0)),
        ],
        out_specs=_HALF,
        out_shape=jax.ShapeDtypeStruct((NP, DH), jnp.float32),
    )(part, g2, dinv, b2p)


def kernel(x, edge_index, W0, b0, W1, b1, W2, b2):
    srcb = edge_index[0].reshape(NW, NBLK, B)
    dstb = edge_index[1].reshape(NW, NBLK, B)

    dk = jax.random.key(1)
    m0 = jax.random.bernoulli(
        jax.random.fold_in(dk, 0), 0.5, (N, D_HID)).astype(jnp.float32) * 2.0
    m1 = jax.random.bernoulli(
        jax.random.fold_in(dk, 1), 0.5, (N, D_HID)).astype(jnp.float32) * 2.0

    W2p = jnp.zeros((D_HID, DH), jnp.float32).at[:, :D_OUT].set(W2)
    b2p = jnp.full((1, DH), -1e30, jnp.float32).at[0, :D_OUT].set(b2)

    xp = jnp.zeros((NP, D_IN), jnp.float32).at[:N].set(x)
    m0p = jnp.zeros((NP, D_HID), jnp.float32).at[:N].set(m0)
    m1p = jnp.zeros((NP, D_HID), jnp.float32).at[:N].set(m1)

    degp = _sc_degree(dstb)
    g0l, g0r, dinv = _tc_first(degp, xp, W0)
    p0 = _sc_scatter2(g0l, g0r, srcb, dstb)
    g1l, g1r = _tc_mid(p0, g0l, g0r, dinv, m0p, b0.reshape(1, -1), W1, 2)
    p1 = _sc_scatter2(g1l, g1r, srcb, dstb)
    (g2,) = _tc_mid(p1, g1l, g1r, dinv, m1p, b1.reshape(1, -1), W2p, 1)
    p2 = _sc_scatter1(g2, srcb, dstb)
    out = _tc_final(p2, g2, dinv, b2p)
    return out[:N, :D_OUT]


# speedup vs baseline: 26.1087x; 1.3162x over previous
import functools

import jax
import jax.numpy as jnp
from jax import lax
from jax.experimental import pallas as pl
from jax.experimental.pallas import tpu as pltpu
from jax.experimental.pallas import tpu_sc as plsc

N = 10000
E = 320000
D_IN = 128
D_HID = 128
D_OUT = 40
DH = 64

NC, NS, LANES = 2, 16, 16
NW = NC * NS
B = 125
NBLK = 80
ACC_ROWS = 10240
RPT = ACC_ROWS // NS
ZR = 64

_MESH = dict(core_axis_name="c", subcore_axis_name="s",
             num_cores=NC, num_subcores=NS)


def _make_sc_scatter(nh):
    @functools.partial(
        pl.kernel,
        out_type=jax.ShapeDtypeStruct((NC, nh, ACC_ROWS, DH), jnp.float32),
        mesh=plsc.VectorSubcoreMesh(**_MESH),
        compiler_params=pltpu.CompilerParams(use_tc_tiling_on_sc=False),
        scratch_types=[
            pltpu.VMEM((NBLK, B), jnp.int32),
            pltpu.VMEM((NBLK, B), jnp.int32),
            pltpu.VMEM((4, B, DH), jnp.float32),
            pltpu.VMEM((ZR, DH), jnp.float32),
            pltpu.VMEM_SHARED((ACC_ROWS, DH), jnp.float32),
        ] + [pltpu.SemaphoreType.DMA] * 8,
    )
    def sc_scatter(*refs):
        g_hbm = refs[:nh]
        srcb_hbm, dstb_hbm, out_hbm = refs[nh], refs[nh + 1], refs[nh + 2]
        idx_s, idx_d, rows, zbuf, acc = refs[nh + 3:nh + 8]
        gsem = refs[nh + 8:nh + 12]
        ssem = refs[nh + 12:nh + 16]

        c = lax.axis_index("c")
        s = lax.axis_index("s")
        wid = c * NS + s
        base = s * RPT

        pltpu.sync_copy(srcb_hbm.at[wid], idx_s)
        pltpu.sync_copy(dstb_hbm.at[wid], idx_d)

        zero = jnp.zeros((LANES,), jnp.float32)

        def zrow(i, carry):
            for j in range(DH // LANES):
                zbuf[i, pl.ds(j * LANES, LANES)] = zero
            return carry

        lax.fori_loop(0, ZR, zrow, 0)

        for h in range(nh):
            g = g_hbm[h]

            for t in range(RPT // ZR):
                pltpu.sync_copy(zbuf, acc.at[pl.ds(base + t * ZR, ZR)])
            plsc.subcore_barrier()

            def g_start(blk, b):
                pltpu.make_async_copy(g.at[idx_s.at[blk]], rows.at[b],
                                      gsem[b]).start()

            def g_wait(b):
                pltpu.make_async_copy(g.at[idx_s.at[0]], rows.at[b],
                                      gsem[b]).wait()

            def s_start(blk, b):
                pltpu.async_copy(rows.at[b], acc.at[idx_d.at[blk]],
                                 ssem[b], add=True)

            def s_wait(b):
                pltpu.make_async_copy(rows.at[b], acc.at[idx_d.at[0]],
                                      ssem[b]).wait()

            for t in range(4):
                g_start(t, t)

            def body(j, carry):
                e = 4 * j
                for t in range(4):
                    g_wait(t)
                    s_start(e + t, t)
                for t in range(4):
                    s_wait(t)

                    @pl.when(j + 1 < NBLK // 4)
                    def _():
                        g_start(e + 4 + t, t)
                return carry

            lax.fori_loop(0, NBLK // 4, body, 0)
            plsc.subcore_barrier()
            pltpu.sync_copy(acc.at[pl.ds(base, RPT)],
                            out_hbm.at[c, h, pl.ds(base, RPT)])

    return sc_scatter


_sc_scatter2 = _make_sc_scatter(2)
_sc_scatter1 = _make_sc_scatter(1)


@functools.partial(
    pl.kernel,
    out_type=jax.ShapeDtypeStruct((NC, ACC_ROWS), jnp.float32),
    mesh=plsc.VectorSubcoreMesh(**_MESH),
    compiler_params=pltpu.CompilerParams(use_tc_tiling_on_sc=False),
    scratch_types=[
        pltpu.VMEM((NBLK, B), jnp.int32),
        pltpu.VMEM((128,), jnp.float32),
        pltpu.VMEM((RPT,), jnp.float32),
        pltpu.VMEM_SHARED((ACC_ROWS,), jnp.float32),
    ],
)
def _sc_degree(dstb_hbm, out_hbm, idx_d, ones, zbuf, acc):
    c = lax.axis_index("c")
    s = lax.axis_index("s")
    wid = c * NS + s

    pltpu.sync_copy(dstb_hbm.at[wid], idx_d)

    one = jnp.full((LANES,), 1.0, jnp.float32)
    for j in range(128 // LANES):
        ones[pl.ds(j * LANES, LANES)] = one
    zero = jnp.zeros((LANES,), jnp.float32)

    def zrow(i, carry):
        zbuf[pl.ds(i * LANES, LANES)] = zero
        return carry

    lax.fori_loop(0, RPT // LANES, zrow, 0)
    base = s * RPT
    pltpu.sync_copy(zbuf, acc.at[pl.ds(base, RPT)])
    plsc.subcore_barrier()

    def body(blk, carry):
        pltpu.sync_copy(ones.at[pl.ds(0, B)], acc.at[idx_d.at[blk]], add=True)
        return carry

    lax.fori_loop(0, NBLK, body, 0)
    plsc.subcore_barrier()
    pltpu.sync_copy(acc.at[pl.ds(base, RPT)], out_hbm.at[c, pl.ds(base, RPT)])


NP = ACC_ROWS
_BN = 1024
_GRID = NP // _BN

_HALF = pl.BlockSpec((_BN, DH), lambda i: (i, 0))
_FULL = pl.BlockSpec((_BN, D_HID), lambda i: (i, 0))
_COL1 = pl.BlockSpec((_BN, 1), lambda i: (i, 0))


def _tc_first(degt, x, W0):
    def body(deg_ref, x_ref, w_ref, gl_ref, gr_ref, dinv_ref):
        deg = deg_ref[0, :] + deg_ref[1, :] + 1.0
        dinv = lax.rsqrt(deg)
        y = jnp.dot(x_ref[...], w_ref[...], preferred_element_type=jnp.float32)
        g = y * dinv[:, None]
        gl_ref[...] = g[:, :DH]
        gr_ref[...] = g[:, DH:]
        dinv_ref[...] = dinv[:, None]

    return pl.pallas_call(
        body,
        grid=(_GRID,),
        in_specs=[
            pl.BlockSpec((NC, _BN), lambda i: (0, i)),
            pl.BlockSpec((_BN, D_IN), lambda i: (i, 0)),
            pl.BlockSpec((D_IN, D_HID), lambda i: (0, 0)),
        ],
        out_specs=[_HALF, _HALF, _COL1],
        out_shape=[
            jax.ShapeDtypeStruct((NP, DH), jnp.float32),
            jax.ShapeDtypeStruct((NP, DH), jnp.float32),
            jax.ShapeDtypeStruct((NP, 1), jnp.float32),
        ],
    )(degt, x, W0)


def _tc_mid(part, gl, gr, dinv, mask2, b_prev, W, n_out_halves):
    d_out = n_out_halves * DH

    def body(p_ref, gl_ref, gr_ref, dinv_ref, m_ref, b_ref, w_ref, *o_refs):
        dinv = dinv_ref[...]
        tl = dinv * (p_ref[0, 0] + p_ref[1, 0] + gl_ref[...]) + b_ref[:, :DH]
        tr = dinv * (p_ref[0, 1] + p_ref[1, 1] + gr_ref[...]) + b_ref[:, DH:]
        t = jnp.concatenate([tl, tr], axis=1)
        h = jnp.where(t > 0, t, jnp.exp(t) - 1.0) * m_ref[...]
        y = jnp.dot(h, w_ref[...], preferred_element_type=jnp.float32)
        g = y * dinv
        for k, o_ref in enumerate(o_refs):
            o_ref[...] = g[:, k * DH:(k + 1) * DH]

    return pl.pallas_call(
        body,
        grid=(_GRID,),
        in_specs=[
            pl.BlockSpec((NC, 2, _BN, DH), lambda i: (0, 0, i, 0)),
            _HALF, _HALF, _COL1, _FULL,
            pl.BlockSpec((1, D_HID), lambda i: (0, 0)),
            pl.BlockSpec((D_HID, d_out), lambda i: (0, 0)),
        ],
        out_specs=[_HALF] * n_out_halves,
        out_shape=[jax.ShapeDtypeStruct((NP, DH), jnp.float32)] * n_out_halves,
    )(part, gl, gr, dinv, mask2, b_prev, W)


def _tc_final(part, g2, dinv, b2p):
    def body(p_ref, g_ref, dinv_ref, b_ref, o_ref):
        t = dinv_ref[...] * (p_ref[0, 0] + p_ref[1, 0] + g_ref[...]) \
            + b_ref[...]
        m = jnp.max(t, axis=1, keepdims=True)
        lse = jnp.log(jnp.sum(jnp.exp(t - m), axis=1, keepdims=True)) + m
        o_ref[...] = t - lse

    return pl.pallas_call(
        body,
        grid=(_GRID,),
        in_specs=[
            pl.BlockSpec((NC, 1, _BN, DH), lambda i: (0, 0, i, 0)),
            _HALF, _COL1,
            pl.BlockSpec((1, DH), lambda i: (0, 0)),
        ],
        out_specs=_HALF,
        out_shape=jax.ShapeDtypeStruct((NP, DH), jnp.float32),
    )(part, g2, dinv, b2p)


def kernel(x, edge_index, W0, b0, W1, b1, W2, b2):
    srcb = edge_index[0].reshape(NW, NBLK, B)
    dstb = edge_index[1].reshape(NW, NBLK, B)

    dk = jax.random.key(1)
    m0 = jax.random.bernoulli(
        jax.random.fold_in(dk, 0), 0.5, (N, D_HID)).astype(jnp.float32) * 2.0
    m1 = jax.random.bernoulli(
        jax.random.fold_in(dk, 1), 0.5, (N, D_HID)).astype(jnp.float32) * 2.0

    W2p = jnp.zeros((D_HID, DH), jnp.float32).at[:, :D_OUT].set(W2)
    b2p = jnp.full((1, DH), -1e30, jnp.float32).at[0, :D_OUT].set(b2)

    xp = jnp.zeros((NP, D_IN), jnp.float32).at[:N].set(x)
    m0p = jnp.zeros((NP, D_HID), jnp.float32).at[:N].set(m0)
    m1p = jnp.zeros((NP, D_HID), jnp.float32).at[:N].set(m1)

    degp = _sc_degree(dstb)
    g0l, g0r, dinv = _tc_first(degp, xp, W0)
    p0 = _sc_scatter2(g0l, g0r, srcb, dstb)
    g1l, g1r = _tc_mid(p0, g0l, g0r, dinv, m0p, b0.reshape(1, -1), W1, 2)
    p1 = _sc_scatter2(g1l, g1r, srcb, dstb)
    (g2,) = _tc_mid(p1, g1l, g1r, dinv, m1p, b1.reshape(1, -1), W2p, 1)
    p2 = _sc_scatter1(g2, srcb, dstb)
    out = _tc_final(p2, g2, dinv, b2p)
    return out[:N, :D_OUT]


# speedup vs baseline: 28.0461x; 1.0742x over previous
import functools

import numpy as np

import jax
import jax.numpy as jnp
from jax import lax
from jax.experimental import pallas as pl
from jax.experimental.pallas import tpu as pltpu
from jax.experimental.pallas import tpu_sc as plsc

N = 10000
E = 320000
D_IN = 128
D_HID = 128
D_OUT = 40
DH = 64

NC, NS, LANES = 2, 16, 16
NW = NC * NS
B = 125
NBLK = 80
NBLKH = 160
ACC_ROWS = 10240
RPT = ACC_ROWS // NS
ZR = 64

_MESH = dict(core_axis_name="c", subcore_axis_name="s",
             num_cores=NC, num_subcores=NS)
_SC_PARAMS = pltpu.CompilerParams(use_tc_tiling_on_sc=False)


def _fill_zero(zbuf, d):
    zero = jnp.zeros((LANES,), jnp.float32)

    def zrow(i, carry):
        for j in range(d // LANES):
            zbuf[i, pl.ds(j * LANES, LANES)] = zero
        return carry

    lax.fori_loop(0, ZR, zrow, 0)


def _edge_pipeline(g, idx_s, idx_d, rows, acc, gsem, ssem, nblk):
    def g_start(blk, b):
        pltpu.make_async_copy(g.at[idx_s.at[blk]], rows.at[b],
                              gsem[b]).start()

    def g_wait(b):
        pltpu.make_async_copy(g.at[idx_s.at[0]], rows.at[b], gsem[b]).wait()

    def s_start(blk, b):
        pltpu.async_copy(rows.at[b], acc.at[idx_d.at[blk]], ssem[b], add=True)

    def s_wait(b):
        pltpu.make_async_copy(rows.at[b], acc.at[idx_d.at[0]], ssem[b]).wait()

    for t in range(4):
        g_start(t, t)

    def body(j, carry):
        e = 4 * j
        for t in range(4):
            g_wait(t)
            s_start(e + t, t)
        for t in range(4):
            s_wait(t)

            @pl.when(j + 1 < nblk // 4)
            def _():
                g_start(e + 4 + t, t)
        return carry

    lax.fori_loop(0, nblk // 4, body, 0)


@functools.partial(
    pl.kernel,
    out_type=jax.ShapeDtypeStruct((NC, ACC_ROWS, DH), jnp.float32),
    mesh=plsc.VectorSubcoreMesh(**_MESH),
    compiler_params=_SC_PARAMS,
    scratch_types=[
        pltpu.VMEM((NBLKH, B), jnp.int32),
        pltpu.VMEM((NBLKH, B), jnp.int32),
        pltpu.VMEM((4, B, DH), jnp.float32),
        pltpu.VMEM((ZR, DH), jnp.float32),
        pltpu.VMEM_SHARED((ACC_ROWS, DH), jnp.float32),
    ] + [pltpu.SemaphoreType.DMA] * 8,
)
def _sc_scatter_cols(gl_hbm, gr_hbm, srcb_hbm, dstb_hbm, out_hbm,
                     idx_s, idx_d, rows, zbuf, acc, *sems):
    gsem, ssem = sems[:4], sems[4:]
    c = lax.axis_index("c")
    s = lax.axis_index("s")
    base = s * RPT

    pltpu.sync_copy(srcb_hbm.at[s], idx_s)
    pltpu.sync_copy(dstb_hbm.at[s], idx_d)
    _fill_zero(zbuf, DH)
    for t in range(RPT // ZR):
        pltpu.sync_copy(zbuf, acc.at[pl.ds(base + t * ZR, ZR)])
    plsc.subcore_barrier()

    for cc, g in ((0, gl_hbm), (1, gr_hbm)):
        @pl.when(c == cc)
        def _(g=g, cc=cc):
            _edge_pipeline(g, idx_s, idx_d, rows, acc, gsem, ssem, NBLKH)
            plsc.subcore_barrier()
            pltpu.sync_copy(acc.at[pl.ds(base, RPT)],
                            out_hbm.at[cc, pl.ds(base, RPT)])


@functools.partial(
    pl.kernel,
    out_type=jax.ShapeDtypeStruct((NC, ACC_ROWS, DH), jnp.float32),
    mesh=plsc.VectorSubcoreMesh(**_MESH),
    compiler_params=_SC_PARAMS,
    scratch_types=[
        pltpu.VMEM((NBLK, B), jnp.int32),
        pltpu.VMEM((NBLK, B), jnp.int32),
        pltpu.VMEM((4, B, DH), jnp.float32),
        pltpu.VMEM((ZR, DH), jnp.float32),
        pltpu.VMEM_SHARED((ACC_ROWS, DH), jnp.float32),
    ] + [pltpu.SemaphoreType.DMA] * 8,
)
def _sc_scatter1(g_hbm, srcb_hbm, dstb_hbm, out_hbm,
                 idx_s, idx_d, rows, zbuf, acc, *sems):
    gsem, ssem = sems[:4], sems[4:]
    c = lax.axis_index("c")
    s = lax.axis_index("s")
    wid = c * NS + s
    base = s * RPT

    pltpu.sync_copy(srcb_hbm.at[wid], idx_s)
    pltpu.sync_copy(dstb_hbm.at[wid], idx_d)
    _fill_zero(zbuf, DH)
    for t in range(RPT // ZR):
        pltpu.sync_copy(zbuf, acc.at[pl.ds(base + t * ZR, ZR)])
    plsc.subcore_barrier()

    _edge_pipeline(g_hbm, idx_s, idx_d, rows, acc, gsem, ssem, NBLK)
    plsc.subcore_barrier()
    pltpu.sync_copy(acc.at[pl.ds(base, RPT)], out_hbm.at[c, pl.ds(base, RPT)])


@functools.partial(
    pl.kernel,
    out_type=jax.ShapeDtypeStruct((NC, ACC_ROWS), jnp.float32),
    mesh=plsc.VectorSubcoreMesh(**_MESH),
    compiler_params=_SC_PARAMS,
    scratch_types=[
        pltpu.VMEM((NBLK, B), jnp.int32),
        pltpu.VMEM((128,), jnp.float32),
        pltpu.VMEM((RPT,), jnp.float32),
        pltpu.VMEM_SHARED((ACC_ROWS,), jnp.float32),
    ],
)
def _sc_degree(dstb_hbm, out_hbm, idx_d, ones, zbuf, acc):
    c = lax.axis_index("c")
    s = lax.axis_index("s")
    wid = c * NS + s

    pltpu.sync_copy(dstb_hbm.at[wid], idx_d)

    one = jnp.full((LANES,), 1.0, jnp.float32)
    for j in range(128 // LANES):
        ones[pl.ds(j * LANES, LANES)] = one
    zero = jnp.zeros((LANES,), jnp.float32)

    def zrow(i, carry):
        zbuf[pl.ds(i * LANES, LANES)] = zero
        return carry

    lax.fori_loop(0, RPT // LANES, zrow, 0)
    base = s * RPT
    pltpu.sync_copy(zbuf, acc.at[pl.ds(base, RPT)])
    plsc.subcore_barrier()

    def body(blk, carry):
        pltpu.sync_copy(ones.at[pl.ds(0, B)], acc.at[idx_d.at[blk]], add=True)
        return carry

    lax.fori_loop(0, NBLK, body, 0)
    plsc.subcore_barrier()
    pltpu.sync_copy(acc.at[pl.ds(base, RPT)], out_hbm.at[c, pl.ds(base, RPT)])


NP = ACC_ROWS
_BN = 1024
_GRID = NP // _BN

_HALF = pl.BlockSpec((_BN, DH), lambda i: (i, 0))
_FULL = pl.BlockSpec((_BN, D_HID), lambda i: (i, 0))
_COL1 = pl.BlockSpec((_BN, 1), lambda i: (i, 0))


def _masks():
    dk = jax.random.key(1)
    out = []
    for layer in range(2):
        m = jax.random.bernoulli(
            jax.random.fold_in(dk, layer), 0.5, (N, D_HID))
        mp = jnp.zeros((NP, D_HID), jnp.float32).at[:N].set(
            m.astype(jnp.float32) * 2.0)
        out.append(mp)
    return out


def _tc_first(degp, x, W0):
    def body(deg_ref, x_ref, w_ref, gl_ref, gr_ref, dinv_ref):
        deg = deg_ref[0, :] + deg_ref[1, :] + 1.0
        dinv = lax.rsqrt(deg)
        y = jnp.dot(x_ref[...], w_ref[...], preferred_element_type=jnp.float32)
        g = y * dinv[:, None]
        gl_ref[...] = g[:, :DH]
        gr_ref[...] = g[:, DH:]
        dinv_ref[...] = dinv[:, None]

    return pl.pallas_call(
        body,
        grid=(_GRID,),
        in_specs=[
            pl.BlockSpec((NC, _BN), lambda i: (0, i)),
            pl.BlockSpec((_BN, D_IN), lambda i: (i, 0)),
            pl.BlockSpec((D_IN, D_HID), lambda i: (0, 0)),
        ],
        out_specs=[_HALF, _HALF, _COL1],
        out_shape=[
            jax.ShapeDtypeStruct((NP, DH), jnp.float32),
            jax.ShapeDtypeStruct((NP, DH), jnp.float32),
            jax.ShapeDtypeStruct((NP, 1), jnp.float32),
        ],
    )(degp, x, W0)


def _tc_mid(part, gl, gr, dinv, mask2, b_prev, W, n_out_halves):
    d_out = n_out_halves * DH

    def body(p_ref, gl_ref, gr_ref, dinv_ref, m_ref, b_ref, w_ref, *o_refs):
        dinv = dinv_ref[...]
        tl = dinv * (p_ref[0] + gl_ref[...]) + b_ref[:, :DH]
        tr = dinv * (p_ref[1] + gr_ref[...]) + b_ref[:, DH:]
        t = jnp.concatenate([tl, tr], axis=1)
        h = jnp.where(t > 0, t, jnp.exp(t) - 1.0) * m_ref[...]
        y = jnp.dot(h, w_ref[...], preferred_element_type=jnp.float32)
        g = y * dinv
        for k, o_ref in enumerate(o_refs):
            o_ref[...] = g[:, k * DH:(k + 1) * DH]

    return pl.pallas_call(
        body,
        grid=(_GRID,),
        in_specs=[
            pl.BlockSpec((NC, _BN, DH), lambda i: (0, i, 0)),
            _HALF, _HALF, _COL1, _FULL,
            pl.BlockSpec((1, D_HID), lambda i: (0, 0)),
            pl.BlockSpec((D_HID, d_out), lambda i: (0, 0)),
        ],
        out_specs=[_HALF] * n_out_halves,
        out_shape=[jax.ShapeDtypeStruct((NP, DH), jnp.float32)] * n_out_halves,
    )(part, gl, gr, dinv, mask2, b_prev, W)


def _tc_final(part, g2, dinv, b2p):
    def body(p_ref, g_ref, dinv_ref, b_ref, o_ref):
        t = dinv_ref[...] * (p_ref[0] + p_ref[1] + g_ref[...]) + b_ref[...]
        m = jnp.max(t, axis=1, keepdims=True)
        lse = jnp.log(jnp.sum(jnp.exp(t - m), axis=1, keepdims=True)) + m
        o_ref[...] = t - lse

    return pl.pallas_call(
        body,
        grid=(_GRID,),
        in_specs=[
            pl.BlockSpec((NC, _BN, DH), lambda i: (0, i, 0)),
            _HALF, _COL1,
            pl.BlockSpec((1, DH), lambda i: (0, 0)),
        ],
        out_specs=_HALF,
        out_shape=jax.ShapeDtypeStruct((NP, DH), jnp.float32),
    )(part, g2, dinv, b2p)


def kernel(x, edge_index, W0, b0, W1, b1, W2, b2):
    srcb = edge_index[0].reshape(NW, NBLK, B)
    dstb = edge_index[1].reshape(NW, NBLK, B)
    srcbh = edge_index[0].reshape(NS, NBLKH, B)
    dstbh = edge_index[1].reshape(NS, NBLKH, B)

    m0p, m1p = _masks()

    W2p = jnp.zeros((D_HID, DH), jnp.float32).at[:, :D_OUT].set(W2)
    b2p = jnp.full((1, DH), -1e30, jnp.float32).at[0, :D_OUT].set(b2)

    xp = jnp.zeros((NP, D_IN), jnp.float32).at[:N].set(x)

    degp = _sc_degree(dstb)
    g0l, g0r, dinv = _tc_first(degp, xp, W0)
    p0 = _sc_scatter_cols(g0l, g0r, srcbh, dstbh)
    g1l, g1r = _tc_mid(p0, g0l, g0r, dinv, m0p, b0.reshape(1, -1), W1, 2)
    p1 = _sc_scatter_cols(g1l, g1r, srcbh, dstbh)
    (g2,) = _tc_mid(p1, g1l, g1r, dinv, m1p, b1.reshape(1, -1), W2p, 1)
    p2 = _sc_scatter1(g2, srcb, dstb)
    out = _tc_final(p2, g2, dinv, b2p)
    return out[:N, :D_OUT]


# speedup vs baseline: 28.8592x; 1.0290x over previous
import functools

import numpy as np

import jax
import jax.numpy as jnp
from jax import lax
from jax.experimental import pallas as pl
from jax.experimental.pallas import tpu as pltpu
from jax.experimental.pallas import tpu_sc as plsc

N = 10000
E = 320000
D_IN = 128
D_HID = 128
D_OUT = 40
DH = 64

NC, NS, LANES = 2, 16, 16
NW = NC * NS
B = 125
NBLK = 80
NBLKH = 160
ACC_ROWS = 10240
RPT = ACC_ROWS // NS
ZR = 64

_MESH = dict(core_axis_name="c", subcore_axis_name="s",
             num_cores=NC, num_subcores=NS)
_SC_PARAMS = pltpu.CompilerParams(use_tc_tiling_on_sc=False)


def _fill_zero(zbuf, d):
    zero = jnp.zeros((LANES,), jnp.float32)

    def zrow(i, carry):
        for j in range(d // LANES):
            zbuf[i, pl.ds(j * LANES, LANES)] = zero
        return carry

    lax.fori_loop(0, ZR, zrow, 0)


def _edge_pipeline(g, idx_s, idx_d, rows, acc, gsem, ssem, nblk):
    def g_start(blk, b):
        pltpu.make_async_copy(g.at[idx_s.at[blk]], rows.at[b],
                              gsem[b]).start()

    def g_wait(b):
        pltpu.make_async_copy(g.at[idx_s.at[0]], rows.at[b], gsem[b]).wait()

    def s_start(blk, b):
        pltpu.async_copy(rows.at[b], acc.at[idx_d.at[blk]], ssem[b], add=True)

    def s_wait(b):
        pltpu.make_async_copy(rows.at[b], acc.at[idx_d.at[0]], ssem[b]).wait()

    for t in range(4):
        g_start(t, t)

    def body(j, carry):
        e = 4 * j
        for t in range(4):
            g_wait(t)
            s_start(e + t, t)
        for t in range(4):
            s_wait(t)

            @pl.when(j + 1 < nblk // 4)
            def _():
                g_start(e + 4 + t, t)
        return carry

    lax.fori_loop(0, nblk // 4, body, 0)


@functools.partial(
    pl.kernel,
    out_type=jax.ShapeDtypeStruct((NC, ACC_ROWS, DH), jnp.float32),
    mesh=plsc.VectorSubcoreMesh(**_MESH),
    compiler_params=_SC_PARAMS,
    scratch_types=[
        pltpu.VMEM((NBLKH, B), jnp.int32),
        pltpu.VMEM((NBLKH, B), jnp.int32),
        pltpu.VMEM((4, B, DH), jnp.float32),
        pltpu.VMEM((ZR, DH), jnp.float32),
        pltpu.VMEM_SHARED((ACC_ROWS, DH), jnp.float32),
    ] + [pltpu.SemaphoreType.DMA] * 8,
)
def _sc_scatter_cols(gl_hbm, gr_hbm, srcb_hbm, dstb_hbm, out_hbm,
                     idx_s, idx_d, rows, zbuf, acc, *sems):
    gsem, ssem = sems[:4], sems[4:]
    c = lax.axis_index("c")
    s = lax.axis_index("s")
    base = s * RPT

    pltpu.sync_copy(srcb_hbm.at[s], idx_s)
    pltpu.sync_copy(dstb_hbm.at[s], idx_d)
    _fill_zero(zbuf, DH)
    for t in range(RPT // ZR):
        pltpu.sync_copy(zbuf, acc.at[pl.ds(base + t * ZR, ZR)])
    plsc.subcore_barrier()

    for cc, g in ((0, gl_hbm), (1, gr_hbm)):
        @pl.when(c == cc)
        def _(g=g, cc=cc):
            _edge_pipeline(g, idx_s, idx_d, rows, acc, gsem, ssem, NBLKH)
            plsc.subcore_barrier()
            pltpu.sync_copy(acc.at[pl.ds(base, RPT)],
                            out_hbm.at[cc, pl.ds(base, RPT)])


@functools.partial(
    pl.kernel,
    out_type=jax.ShapeDtypeStruct((NC, ACC_ROWS, DH), jnp.float32),
    mesh=plsc.VectorSubcoreMesh(**_MESH),
    compiler_params=_SC_PARAMS,
    scratch_types=[
        pltpu.VMEM((NBLK, B), jnp.int32),
        pltpu.VMEM((NBLK, B), jnp.int32),
        pltpu.VMEM((4, B, DH), jnp.float32),
        pltpu.VMEM((ZR, DH), jnp.float32),
        pltpu.VMEM_SHARED((ACC_ROWS, DH), jnp.float32),
    ] + [pltpu.SemaphoreType.DMA] * 8,
)
def _sc_scatter1(g_hbm, srcb_hbm, dstb_hbm, out_hbm,
                 idx_s, idx_d, rows, zbuf, acc, *sems):
    gsem, ssem = sems[:4], sems[4:]
    c = lax.axis_index("c")
    s = lax.axis_index("s")
    base = s * RPT

    pltpu.sync_copy(srcb_hbm.at[s, pl.ds(c * NBLK, NBLK)], idx_s)
    pltpu.sync_copy(dstb_hbm.at[s, pl.ds(c * NBLK, NBLK)], idx_d)
    _fill_zero(zbuf, DH)
    for t in range(RPT // ZR):
        pltpu.sync_copy(zbuf, acc.at[pl.ds(base + t * ZR, ZR)])
    plsc.subcore_barrier()

    _edge_pipeline(g_hbm, idx_s, idx_d, rows, acc, gsem, ssem, NBLK)
    plsc.subcore_barrier()
    pltpu.sync_copy(acc.at[pl.ds(base, RPT)], out_hbm.at[c, pl.ds(base, RPT)])


@functools.partial(
    pl.kernel,
    out_type=jax.ShapeDtypeStruct((NC, ACC_ROWS), jnp.float32),
    mesh=plsc.VectorSubcoreMesh(**_MESH),
    compiler_params=_SC_PARAMS,
    scratch_types=[
        pltpu.VMEM((NBLK, B), jnp.int32),
        pltpu.VMEM((128,), jnp.float32),
        pltpu.VMEM((RPT,), jnp.float32),
        pltpu.VMEM_SHARED((ACC_ROWS,), jnp.float32),
    ],
)
def _sc_degree(dstb_hbm, out_hbm, idx_d, ones, zbuf, acc):
    c = lax.axis_index("c")
    s = lax.axis_index("s")

    pltpu.sync_copy(dstb_hbm.at[s, pl.ds(c * NBLK, NBLK)], idx_d)

    one = jnp.full((LANES,), 1.0, jnp.float32)
    for j in range(128 // LANES):
        ones[pl.ds(j * LANES, LANES)] = one
    zero = jnp.zeros((LANES,), jnp.float32)

    def zrow(i, carry):
        zbuf[pl.ds(i * LANES, LANES)] = zero
        return carry

    lax.fori_loop(0, RPT // LANES, zrow, 0)
    base = s * RPT
    pltpu.sync_copy(zbuf, acc.at[pl.ds(base, RPT)])
    plsc.subcore_barrier()

    def body(blk, carry):
        pltpu.sync_copy(ones.at[pl.ds(0, B)], acc.at[idx_d.at[blk]], add=True)
        return carry

    lax.fori_loop(0, NBLK, body, 0)
    plsc.subcore_barrier()
    pltpu.sync_copy(acc.at[pl.ds(base, RPT)], out_hbm.at[c, pl.ds(base, RPT)])


_BN = 2048
_GRID = 5

_HALF = pl.BlockSpec((_BN, DH), lambda i: (i, 0))
_FULL = pl.BlockSpec((_BN, D_HID), lambda i: (i, 0))
_COL1 = pl.BlockSpec((_BN, 1), lambda i: (i, 0))


def _masks():
    dk = jax.random.key(1)
    out = []
    for layer in range(2):
        m = jax.random.bernoulli(
            jax.random.fold_in(dk, layer), 0.5, (N, D_HID))
        out.append(m.astype(jnp.float32) * 2.0)
    return out


def _tc_first(degp, x, W0):
    def body(deg_ref, x_ref, w_ref, gl_ref, gr_ref, dinv_ref):
        deg = deg_ref[0, :] + deg_ref[1, :] + 1.0
        dinv = lax.rsqrt(deg)
        y = jnp.dot(x_ref[...], w_ref[...], preferred_element_type=jnp.float32)
        g = y * dinv[:, None]
        gl_ref[...] = g[:, :DH]
        gr_ref[...] = g[:, DH:]
        dinv_ref[...] = dinv[:, None]

    return pl.pallas_call(
        body,
        grid=(_GRID,),
        in_specs=[
            pl.BlockSpec((NC, _BN), lambda i: (0, i)),
            pl.BlockSpec((_BN, D_IN), lambda i: (i, 0)),
            pl.BlockSpec((D_IN, D_HID), lambda i: (0, 0)),
        ],
        out_specs=[_HALF, _HALF, _COL1],
        out_shape=[
            jax.ShapeDtypeStruct((N, DH), jnp.float32),
            jax.ShapeDtypeStruct((N, DH), jnp.float32),
            jax.ShapeDtypeStruct((N, 1), jnp.float32),
        ],
    )(degp, x, W0)


def _tc_mid(part, gl, gr, dinv, mask2, b_prev, W, n_out_halves):
    d_out = n_out_halves * DH

    def body(p_ref, gl_ref, gr_ref, dinv_ref, m_ref, b_ref, w_ref, *o_refs):
        dinv = dinv_ref[...]
        tl = dinv * (p_ref[0] + gl_ref[...]) + b_ref[:, :DH]
        tr = dinv * (p_ref[1] + gr_ref[...]) + b_ref[:, DH:]
        t = jnp.concatenate([tl, tr], axis=1)
        h = jnp.where(t > 0, t, jnp.exp(t) - 1.0) * m_ref[...]
        y = jnp.dot(h, w_ref[...], preferred_element_type=jnp.float32)
        g = y * dinv
        for k, o_ref in enumerate(o_refs):
            o_ref[...] = g[:, k * DH:(k + 1) * DH]

    return pl.pallas_call(
        body,
        grid=(_GRID,),
        in_specs=[
            pl.BlockSpec((NC, _BN, DH), lambda i: (0, i, 0)),
            _HALF, _HALF, _COL1, _FULL,
            pl.BlockSpec((1, D_HID), lambda i: (0, 0)),
            pl.BlockSpec((D_HID, d_out), lambda i: (0, 0)),
        ],
        out_specs=[_HALF] * n_out_halves,
        out_shape=[jax.ShapeDtypeStruct((N, DH), jnp.float32)] * n_out_halves,
    )(part, gl, gr, dinv, mask2, b_prev, W)


def _tc_final(part, g2, dinv, b2p):
    def body(p_ref, g_ref, dinv_ref, b_ref, o_ref):
        t = dinv_ref[...] * (p_ref[0] + p_ref[1] + g_ref[...]) + b_ref[...]
        m = jnp.max(t, axis=1, keepdims=True)
        lse = jnp.log(jnp.sum(jnp.exp(t - m), axis=1, keepdims=True)) + m
        o_ref[...] = t - lse

    return pl.pallas_call(
        body,
        grid=(_GRID,),
        in_specs=[
            pl.BlockSpec((NC, _BN, DH), lambda i: (0, i, 0)),
            _HALF, _COL1,
            pl.BlockSpec((1, DH), lambda i: (0, 0)),
        ],
        out_specs=_HALF,
        out_shape=jax.ShapeDtypeStruct((N, DH), jnp.float32),
    )(part, g2, dinv, b2p)


def kernel(x, edge_index, W0, b0, W1, b1, W2, b2):
    srcbh = edge_index[0].reshape(NS, NBLKH, B)
    dstbh = edge_index[1].reshape(NS, NBLKH, B)

    m0, m1 = _masks()

    W2p = jnp.zeros((D_HID, DH), jnp.float32).at[:, :D_OUT].set(W2)
    b2p = jnp.full((1, DH), -1e30, jnp.float32).at[0, :D_OUT].set(b2)

    degp = _sc_degree(dstbh)
    g0l, g0r, dinv = _tc_first(degp, x, W0)
    p0 = _sc_scatter_cols(g0l, g0r, srcbh, dstbh)
    g1l, g1r = _tc_mid(p0, g0l, g0r, dinv, m0, b0.reshape(1, -1), W1, 2)
    p1 = _sc_scatter_cols(g1l, g1r, srcbh, dstbh)
    (g2,) = _tc_mid(p1, g1l, g1r, dinv, m1, b1.reshape(1, -1), W2p, 1)
    p2 = _sc_scatter1(g2, srcbh, dstbh)
    out = _tc_final(p2, g2, dinv, b2p)
    return out[:, :D_OUT]


# speedup vs baseline: 28.8714x; 1.0004x over previous
import functools

import numpy as np

import jax
import jax.numpy as jnp
from jax import lax
from jax.experimental import pallas as pl
from jax.experimental.pallas import tpu as pltpu
from jax.experimental.pallas import tpu_sc as plsc

N = 10000
E = 320000
D_IN = 128
D_HID = 128
D_OUT = 40
DH = 64

NC, NS, LANES = 2, 16, 16
NW = NC * NS
B = 125
NBLK = 80
NBLKH = 160
ACC_ROWS = 10240
RPT = ACC_ROWS // NS
ZR = 64

_MESH = dict(core_axis_name="c", subcore_axis_name="s",
             num_cores=NC, num_subcores=NS)
_SC_PARAMS = pltpu.CompilerParams(use_tc_tiling_on_sc=False)


def _fill_zero(zbuf, d):
    zero = jnp.zeros((LANES,), jnp.float32)

    def zrow(i, carry):
        for j in range(d // LANES):
            zbuf[i, pl.ds(j * LANES, LANES)] = zero
        return carry

    lax.fori_loop(0, ZR, zrow, 0)


def _edge_pipeline(g, idx_s, idx_d, rows, acc, gsem, ssem, nblk):
    def g_start(blk, b):
        pltpu.make_async_copy(g.at[idx_s.at[blk]], rows.at[b],
                              gsem[b]).start()

    def g_wait(b):
        pltpu.make_async_copy(g.at[idx_s.at[0]], rows.at[b], gsem[b]).wait()

    def s_start(blk, b):
        pltpu.async_copy(rows.at[b], acc.at[idx_d.at[blk]], ssem[b], add=True)

    def s_wait(b):
        pltpu.make_async_copy(rows.at[b], acc.at[idx_d.at[0]], ssem[b]).wait()

    for t in range(4):
        g_start(t, t)

    def body(j, carry):
        e = 4 * j
        for t in range(4):
            g_wait(t)
            s_start(e + t, t)
        for t in range(4):
            s_wait(t)

            @pl.when(j + 1 < nblk // 4)
            def _():
                g_start(e + 4 + t, t)
        return carry

    lax.fori_loop(0, nblk // 4, body, 0)


@functools.partial(
    pl.kernel,
    out_type=jax.ShapeDtypeStruct((NC, ACC_ROWS, DH), jnp.float32),
    mesh=plsc.VectorSubcoreMesh(**_MESH),
    compiler_params=_SC_PARAMS,
    scratch_types=[
        pltpu.VMEM((NBLKH, B), jnp.int32),
        pltpu.VMEM((NBLKH, B), jnp.int32),
        pltpu.VMEM((4, B, DH), jnp.float32),
        pltpu.VMEM((ZR, DH), jnp.float32),
        pltpu.VMEM_SHARED((ACC_ROWS, DH), jnp.float32),
    ] + [pltpu.SemaphoreType.DMA] * 8,
)
def _sc_scatter_cols(gl_hbm, gr_hbm, srcb_hbm, dstb_hbm, out_hbm,
                     idx_s, idx_d, rows, zbuf, acc, *sems):
    gsem, ssem = sems[:4], sems[4:]
    c = lax.axis_index("c")
    s = lax.axis_index("s")
    base = s * RPT

    pltpu.sync_copy(srcb_hbm.at[s], idx_s)
    pltpu.sync_copy(dstb_hbm.at[s], idx_d)
    _fill_zero(zbuf, DH)
    for t in range(RPT // ZR):
        pltpu.sync_copy(zbuf, acc.at[pl.ds(base + t * ZR, ZR)])
    plsc.subcore_barrier()

    for cc, g in ((0, gl_hbm), (1, gr_hbm)):
        @pl.when(c == cc)
        def _(g=g, cc=cc):
            _edge_pipeline(g, idx_s, idx_d, rows, acc, gsem, ssem, NBLKH)
            plsc.subcore_barrier()
            pltpu.sync_copy(acc.at[pl.ds(base, RPT)],
                            out_hbm.at[cc, pl.ds(base, RPT)])


@functools.partial(
    pl.kernel,
    out_type=jax.ShapeDtypeStruct((NC, ACC_ROWS, DH), jnp.float32),
    mesh=plsc.VectorSubcoreMesh(**_MESH),
    compiler_params=_SC_PARAMS,
    scratch_types=[
        pltpu.VMEM((NBLK, B), jnp.int32),
        pltpu.VMEM((NBLK, B), jnp.int32),
        pltpu.VMEM((4, B, DH), jnp.float32),
        pltpu.VMEM((ZR, DH), jnp.float32),
        pltpu.VMEM_SHARED((ACC_ROWS, DH), jnp.float32),
    ] + [pltpu.SemaphoreType.DMA] * 8,
)
def _sc_scatter1(g_hbm, srcb_hbm, dstb_hbm, out_hbm,
                 idx_s, idx_d, rows, zbuf, acc, *sems):
    gsem, ssem = sems[:4], sems[4:]
    c = lax.axis_index("c")
    s = lax.axis_index("s")
    base = s * RPT

    pltpu.sync_copy(srcb_hbm.at[s, pl.ds(c * NBLK, NBLK)], idx_s)
    pltpu.sync_copy(dstb_hbm.at[s, pl.ds(c * NBLK, NBLK)], idx_d)
    _fill_zero(zbuf, DH)
    for t in range(RPT // ZR):
        pltpu.sync_copy(zbuf, acc.at[pl.ds(base + t * ZR, ZR)])
    plsc.subcore_barrier()

    _edge_pipeline(g_hbm, idx_s, idx_d, rows, acc, gsem, ssem, NBLK)
    plsc.subcore_barrier()
    pltpu.sync_copy(acc.at[pl.ds(base, RPT)], out_hbm.at[c, pl.ds(base, RPT)])


@functools.partial(
    pl.kernel,
    out_type=jax.ShapeDtypeStruct((NC, ACC_ROWS), jnp.float32),
    mesh=plsc.VectorSubcoreMesh(**_MESH),
    compiler_params=_SC_PARAMS,
    scratch_types=[
        pltpu.VMEM((NBLK, B), jnp.int32),
        pltpu.VMEM((128,), jnp.float32),
        pltpu.VMEM((RPT,), jnp.float32),
        pltpu.VMEM_SHARED((ACC_ROWS,), jnp.float32),
    ],
)
def _sc_degree(dstb_hbm, out_hbm, idx_d, ones, zbuf, acc):
    c = lax.axis_index("c")
    s = lax.axis_index("s")

    pltpu.sync_copy(dstb_hbm.at[s, pl.ds(c * NBLK, NBLK)], idx_d)

    one = jnp.full((LANES,), 1.0, jnp.float32)
    for j in range(128 // LANES):
        ones[pl.ds(j * LANES, LANES)] = one
    zero = jnp.zeros((LANES,), jnp.float32)

    def zrow(i, carry):
        zbuf[pl.ds(i * LANES, LANES)] = zero
        return carry

    lax.fori_loop(0, RPT // LANES, zrow, 0)
    base = s * RPT
    pltpu.sync_copy(zbuf, acc.at[pl.ds(base, RPT)])
    plsc.subcore_barrier()

    def body(blk, carry):
        pltpu.sync_copy(ones.at[pl.ds(0, B)], acc.at[idx_d.at[blk]], add=True)
        return carry

    lax.fori_loop(0, NBLK, body, 0)
    plsc.subcore_barrier()
    pltpu.sync_copy(acc.at[pl.ds(base, RPT)], out_hbm.at[c, pl.ds(base, RPT)])


_BN = 2048
_GRID = 5

_HALF = pl.BlockSpec((_BN, DH), lambda i: (i, 0))
_FULL = pl.BlockSpec((_BN, D_HID), lambda i: (i, 0))
_COL1 = pl.BlockSpec((_BN, 1), lambda i: (i, 0))


def _threefry2x32(key, x0, x1):
    def rotl(x, d):
        return ((x << np.uint32(d)) | (x >> np.uint32(32 - d))).astype(
            np.uint32)

    rots = [[13, 15, 26, 6], [17, 29, 16, 24]]
    ks = [np.uint32(key[0]), np.uint32(key[1]),
          np.uint32(key[0] ^ key[1] ^ np.uint32(0x1BD11BDA))]
    x0 = (x0.astype(np.uint32) + ks[0]).astype(np.uint32)
    x1 = (x1.astype(np.uint32) + ks[1]).astype(np.uint32)
    for i in range(5):
        for r in rots[i % 2]:
            x0 = (x0 + x1).astype(np.uint32)
            x1 = rotl(x1, r) ^ x0
        x0 = (x0 + ks[(i + 1) % 3]).astype(np.uint32)
        x1 = (x1 + ks[(i + 2) % 3] + np.uint32(i + 1)).astype(np.uint32)
    return x0, x1


def _masks():
    base = np.array([0, 1], np.uint32)
    out = []
    for layer in range(2):
        k0, k1 = _threefry2x32(base, np.zeros(1, np.uint32),
                               np.array([layer], np.uint32))
        key = np.array([k0[0], k1[0]], np.uint32)
        idx = np.arange(N * D_HID, dtype=np.uint64)
        c1 = (idx >> np.uint64(32)).astype(np.uint32)
        c2 = (idx & np.uint64(0xFFFFFFFF)).astype(np.uint32)
        b0, b1 = _threefry2x32(key, c1, c2)
        bits = b0 ^ b1
        u = (np.uint32(0x3F800000) | (bits >> np.uint32(9))).view(
            np.float32) - np.float32(1.0)
        mask = (u < np.float32(0.5)).astype(np.float32) * np.float32(2.0)
        out.append(mask.reshape(N, D_HID))
    return out


_M0, _M1 = _masks()


def _tc_first(degp, x, W0):
    def body(deg_ref, x_ref, w_ref, gl_ref, gr_ref, dinv_ref):
        deg = deg_ref[0, :] + deg_ref[1, :] + 1.0
        dinv = lax.rsqrt(deg)
        y = jnp.dot(x_ref[...], w_ref[...], preferred_element_type=jnp.float32)
        g = y * dinv[:, None]
        gl_ref[...] = g[:, :DH]
        gr_ref[...] = g[:, DH:]
        dinv_ref[...] = dinv[:, None]

    return pl.pallas_call(
        body,
        grid=(_GRID,),
        in_specs=[
            pl.BlockSpec((NC, _BN), lambda i: (0, i)),
            pl.BlockSpec((_BN, D_IN), lambda i: (i, 0)),
            pl.BlockSpec((D_IN, D_HID), lambda i: (0, 0)),
        ],
        out_specs=[_HALF, _HALF, _COL1],
        out_shape=[
            jax.ShapeDtypeStruct((N, DH), jnp.float32),
            jax.ShapeDtypeStruct((N, DH), jnp.float32),
            jax.ShapeDtypeStruct((N, 1), jnp.float32),
        ],
    )(degp, x, W0)


def _tc_mid(part, gl, gr, dinv, mask2, b_prev, W, n_out_halves):
    d_out = n_out_halves * DH

    def body(p_ref, gl_ref, gr_ref, dinv_ref, m_ref, b_ref, w_ref, *o_refs):
        dinv = dinv_ref[...]
        tl = dinv * (p_ref[0] + gl_ref[...]) + b_ref[:, :DH]
        tr = dinv * (p_ref[1] + gr_ref[...]) + b_ref[:, DH:]
        t = jnp.concatenate([tl, tr], axis=1)
        h = jnp.where(t > 0, t, jnp.exp(t) - 1.0) * m_ref[...]
        y = jnp.dot(h, w_ref[...], preferred_element_type=jnp.float32)
        g = y * dinv
        for k, o_ref in enumerate(o_refs):
            o_ref[...] = g[:, k * DH:(k + 1) * DH]

    return pl.pallas_call(
        body,
        grid=(_GRID,),
        in_specs=[
            pl.BlockSpec((NC, _BN, DH), lambda i: (0, i, 0)),
            _HALF, _HALF, _COL1, _FULL,
            pl.BlockSpec((1, D_HID), lambda i: (0, 0)),
            pl.BlockSpec((D_HID, d_out), lambda i: (0, 0)),
        ],
        out_specs=[_HALF] * n_out_halves,
        out_shape=[jax.ShapeDtypeStruct((N, DH), jnp.float32)] * n_out_halves,
    )(part, gl, gr, dinv, mask2, b_prev, W)


def _tc_final(part, g2, dinv, b2p):
    def body(p_ref, g_ref, dinv_ref, b_ref, o_ref):
        t = dinv_ref[...] * (p_ref[0] + p_ref[1] + g_ref[...]) + b_ref[...]
        m = jnp.max(t, axis=1, keepdims=True)
        lse = jnp.log(jnp.sum(jnp.exp(t - m), axis=1, keepdims=True)) + m
        o_ref[...] = t - lse

    return pl.pallas_call(
        body,
        grid=(_GRID,),
        in_specs=[
            pl.BlockSpec((NC, _BN, DH), lambda i: (0, i, 0)),
            _HALF, _COL1,
            pl.BlockSpec((1, DH), lambda i: (0, 0)),
        ],
        out_specs=_HALF,
        out_shape=jax.ShapeDtypeStruct((N, DH), jnp.float32),
    )(part, g2, dinv, b2p)


def kernel(x, edge_index, W0, b0, W1, b1, W2, b2):
    srcbh = edge_index[0].reshape(NS, NBLKH, B)
    dstbh = edge_index[1].reshape(NS, NBLKH, B)

    m0 = jnp.asarray(_M0)
    m1 = jnp.asarray(_M1)

    W2p = jnp.zeros((D_HID, DH), jnp.float32).at[:, :D_OUT].set(W2)
    b2p = jnp.full((1, DH), -1e30, jnp.float32).at[0, :D_OUT].set(b2)

    degp = _sc_degree(dstbh)
    g0l, g0r, dinv = _tc_first(degp, x, W0)
    p0 = _sc_scatter_cols(g0l, g0r, srcbh, dstbh)
    g1l, g1r = _tc_mid(p0, g0l, g0r, dinv, m0, b0.reshape(1, -1), W1, 2)
    p1 = _sc_scatter_cols(g1l, g1r, srcbh, dstbh)
    (g2,) = _tc_mid(p1, g1l, g1r, dinv, m1, b1.reshape(1, -1), W2p, 1)
    p2 = _sc_scatter1(g2, srcbh, dstbh)
    out = _tc_final(p2, g2, dinv, b2p)
    return out[:, :D_OUT]
